# baseline (speedup 1.0000x reference)
import jax
import jax.numpy as jnp
from jax.experimental import pallas as pl
from jax.experimental.pallas import tpu as pltpu

Q = 1024
K = 100000
D = 128
TOPK = 8
BQ = 256
BK = 2048
KPAD = 100352
NKT = KPAD // BK


def _knn_kernel(q_ref, k_ref, out_s_ref, out_i_ref, qn_ref, run_s_ref, run_i_ref):
    j = pl.program_id(1)

    @pl.when(j == 0)
    def _init():
        q = q_ref[...]
        qn = jnp.sqrt(jnp.sum(q * q, axis=1, keepdims=True))
        qn_ref[...] = q / (qn + 1e-8)
        run_s_ref[...] = jnp.full((BQ, TOPK), -jnp.inf, dtype=jnp.float32)
        run_i_ref[...] = jax.lax.broadcasted_iota(jnp.int32, (BQ, TOPK), 1)

    k = k_ref[...]
    kn = jnp.sqrt(jnp.sum(k * k, axis=1, keepdims=True))
    kn = k / (kn + 1e-8)

    sims = jax.lax.dot_general(
        qn_ref[...], kn, (((1,), (1,)), ((), ())),
        preferred_element_type=jnp.float32,
        precision=jax.lax.Precision.DEFAULT,
    )

    col = jax.lax.broadcasted_iota(jnp.int32, (BQ, BK), 1) + j * BK
    sims = jnp.where(col < K, sims, -jnp.inf)

    lane = jax.lax.broadcasted_iota(jnp.int32, (BQ, BK), 1)
    tile_s = []
    tile_i = []
    for _ in range(TOPK):
        a = jnp.argmax(sims, axis=1)
        m = jnp.max(sims, axis=1)
        tile_s.append(m)
        tile_i.append(a + j * BK)
        sims = jnp.where(lane == a[:, None], -jnp.inf, sims)
    tile_s = jnp.stack(tile_s, axis=1)
    tile_i = jnp.stack(tile_i, axis=1)

    cand_s = jnp.concatenate([run_s_ref[...], tile_s], axis=1)
    cand_i = jnp.concatenate([run_i_ref[...], tile_i], axis=1)

    rank_cols = []
    for t in range(2 * TOPK):
        s_t = cand_s[:, t:t + 1]
        i_t = cand_i[:, t:t + 1]
        better = (cand_s > s_t) | ((cand_s == s_t) & (cand_i < i_t))
        rank_cols.append(jnp.sum(better.astype(jnp.int32), axis=1))
    rank = jnp.stack(rank_cols, axis=1)

    new_s = []
    new_i = []
    for t in range(TOPK):
        sel = rank == t
        new_s.append(jnp.sum(jnp.where(sel, cand_s, 0.0), axis=1))
        new_i.append(jnp.sum(jnp.where(sel, cand_i, 0), axis=1))
    run_s_ref[...] = jnp.stack(new_s, axis=1)
    run_i_ref[...] = jnp.stack(new_i, axis=1)

    @pl.when(j == NKT - 1)
    def _emit():
        out_s_ref[...] = run_s_ref[...]
        out_i_ref[...] = run_i_ref[...]


@jax.jit
def kernel(queries, keys):
    keys_p = jnp.pad(keys, ((0, KPAD - K), (0, 0)))
    grid = (Q // BQ, NKT)
    out_s, out_i = pl.pallas_call(
        _knn_kernel,
        grid=grid,
        in_specs=[
            pl.BlockSpec((BQ, D), lambda i, j: (i, 0)),
            pl.BlockSpec((BK, D), lambda i, j: (j, 0)),
        ],
        out_specs=[
            pl.BlockSpec((BQ, TOPK), lambda i, j: (i, 0)),
            pl.BlockSpec((BQ, TOPK), lambda i, j: (i, 0)),
        ],
        out_shape=[
            jax.ShapeDtypeStruct((Q, TOPK), jnp.float32),
            jax.ShapeDtypeStruct((Q, TOPK), jnp.int32),
        ],
        scratch_shapes=[
            pltpu.VMEM((BQ, D), jnp.float32),
            pltpu.VMEM((BQ, TOPK), jnp.float32),
            pltpu.VMEM((BQ, TOPK), jnp.int32),
        ],
    )(queries, keys_p)
    return out_s, out_i


# speedup vs baseline: 1.4570x; 1.4570x over previous
import jax
import jax.numpy as jnp
from jax.experimental import pallas as pl
from jax.experimental.pallas import tpu as pltpu

Q = 1024
K = 100000
D = 128
TOPK = 8
BQ = 256
BK = 2048
KPAD = 100352
NKT = KPAD // BK
IBIG = 2**30


def _knn_kernel(q_ref, k_ref, qn2_ref, kn2_ref, out_s_ref, out_i_ref,
                qn_ref, buf_s_ref, buf_i_ref, tile_s_ref, tile_i_ref):
    j = pl.program_id(1)

    @pl.when(j == 0)
    def _init():
        qn_ref[...] = q_ref[...] / qn2_ref[...]

    kn = k_ref[...] / kn2_ref[...]

    sims = jax.lax.dot_general(
        qn_ref[...], kn, (((1,), (1,)), ((), ())),
        preferred_element_type=jnp.float32,
        precision=jax.lax.Precision.DEFAULT,
    )

    col = jax.lax.broadcasted_iota(jnp.int32, (BQ, BK), 1) + j * BK
    sims = jnp.where(col < K, sims, -jnp.inf)

    for t in range(TOPK):
        m = jnp.max(sims, axis=1)
        a = jnp.min(jnp.where(sims == m[:, None], col, IBIG), axis=1)
        tile_s_ref[:, 0, t] = m
        tile_i_ref[:, 0, t] = a
        sims = jnp.where(col == a[:, None], -jnp.inf, sims)

    buf_s_ref[:, pl.ds(j, 1), :] = tile_s_ref[...]
    buf_i_ref[:, pl.ds(j, 1), :] = tile_i_ref[...]

    @pl.when(j == NKT - 1)
    def _emit():
        s = buf_s_ref[...]
        i = buf_i_ref[...]
        for t in range(TOPK):
            m = jnp.max(s, axis=(1, 2))
            a = jnp.min(jnp.where(s == m[:, None, None], i, IBIG), axis=(1, 2))
            out_s_ref[:, t] = m
            out_i_ref[:, t] = a
            s = jnp.where(i == a[:, None, None], -jnp.inf, s)


@jax.jit
def kernel(queries, keys):
    keys_p = jnp.pad(keys, ((0, KPAD - K), (0, 0)))
    qnorm = jnp.linalg.norm(queries, axis=-1, keepdims=True) + 1e-8
    knorm = jnp.linalg.norm(keys_p, axis=-1, keepdims=True) + 1e-8
    grid = (Q // BQ, NKT)
    out_s, out_i = pl.pallas_call(
        _knn_kernel,
        grid=grid,
        in_specs=[
            pl.BlockSpec((BQ, D), lambda i, j: (i, 0)),
            pl.BlockSpec((BK, D), lambda i, j: (j, 0)),
            pl.BlockSpec((BQ, 1), lambda i, j: (i, 0)),
            pl.BlockSpec((BK, 1), lambda i, j: (j, 0)),
        ],
        out_specs=[
            pl.BlockSpec((BQ, TOPK), lambda i, j: (i, 0)),
            pl.BlockSpec((BQ, TOPK), lambda i, j: (i, 0)),
        ],
        out_shape=[
            jax.ShapeDtypeStruct((Q, TOPK), jnp.float32),
            jax.ShapeDtypeStruct((Q, TOPK), jnp.int32),
        ],
        scratch_shapes=[
            pltpu.VMEM((BQ, D), jnp.float32),
            pltpu.VMEM((BQ, NKT, TOPK), jnp.float32),
            pltpu.VMEM((BQ, NKT, TOPK), jnp.int32),
            pltpu.VMEM((BQ, 1, TOPK), jnp.float32),
            pltpu.VMEM((BQ, 1, TOPK), jnp.int32),
        ],
    )(queries, keys_p, qnorm, knorm)
    return out_s, out_i


# speedup vs baseline: 1.7308x; 1.1879x over previous
import jax
import jax.numpy as jnp
from jax.experimental import pallas as pl
from jax.experimental.pallas import tpu as pltpu

Q = 1024
K = 100000
D = 128
TOPK = 8
BQ = 256
BK = 2048
KPAD = 100352
NKT = KPAD // BK
IBIG = 2**30
NEG = -jnp.inf


def _tree_max(s):
    w = s.shape[1]
    while w > 128:
        h = w // 2
        s = jnp.maximum(s[:, :h], s[:, h:w])
        w = h
    return jnp.max(s, axis=1)


def _tree_max_arg(s, col):
    w = s.shape[1]
    while w > 128:
        h = w // 2
        lo, hi = s[:, :h], s[:, h:w]
        cl, ch = col[:, :h], col[:, h:w]
        cmp = lo >= hi
        s = jnp.where(cmp, lo, hi)
        col = jnp.where(cmp, cl, ch)
        w = h
    m = jnp.max(s, axis=1)
    a = jnp.min(jnp.where(s == m[:, None], col, IBIG), axis=1)
    return m, a


def _knn_kernel(q_ref, k_ref, qn2_ref, kn2_ref, out_s_ref, out_i_ref,
                qn_ref, sims_ref, tmax_ref, tau_ref, buf_s_ref, buf_i_ref):
    sweep = pl.program_id(1)
    j = pl.program_id(2)

    @pl.when((sweep == 0) & (j == 0))
    def _init():
        qn_ref[...] = q_ref[...] / qn2_ref[...]

    kn = k_ref[...] / kn2_ref[...]
    sims = jax.lax.dot_general(
        qn_ref[...], kn, (((1,), (1,)), ((), ())),
        preferred_element_type=jnp.float32,
        precision=jax.lax.Precision.DEFAULT,
    )
    col = jax.lax.broadcasted_iota(jnp.int32, (BQ, BK), 1) + j * BK
    sims = jnp.where(col < K, sims, NEG)

    @pl.when(sweep == 0)
    def _sweep1():
        tmax_ref[:, pl.ds(j, 1), :] = _tree_max(sims)[:, None, None]

    @pl.when(sweep == 1)
    def _sweep2():
        @pl.when(j == 0)
        def _tau():
            v = tmax_ref[...]
            for _ in range(TOPK - 1):
                mv = jnp.max(v, axis=(1, 2))
                v = jnp.where(v == mv[:, None, None], NEG, v)
            tau_ref[...] = jnp.max(v, axis=(1, 2))[:, None]

        sims_ref[...] = sims
        tau = tau_ref[...][:, 0]
        m0, a0 = _tree_max_arg(sims, col)
        lane8 = jax.lax.broadcasted_iota(jnp.int32, (BQ, TOPK), 1)
        cand_s0 = jnp.full((BQ, TOPK), NEG, dtype=jnp.float32)
        cand_i0 = jnp.full((BQ, TOPK), IBIG, dtype=jnp.int32)

        def cond(carry):
            t, _, _, m, _ = carry
            return (t < TOPK) & jnp.any(m >= tau)

        def body(carry):
            t, cand_s, cand_i, m, a = carry
            upd = (m >= tau)[:, None] & (lane8 == t)
            cand_s = jnp.where(upd, m[:, None], cand_s)
            cand_i = jnp.where(upd, a[:, None], cand_i)
            sims_ref[...] = jnp.where(col == a[:, None], NEG, sims_ref[...])
            m, a = _tree_max_arg(sims_ref[...], col)
            return t + 1, cand_s, cand_i, m, a

        _, cand_s, cand_i, _, _ = jax.lax.while_loop(
            cond, body, (jnp.int32(0), cand_s0, cand_i0, m0, a0))
        buf_s_ref[:, pl.ds(j, 1), :] = cand_s[:, None, :]
        buf_i_ref[:, pl.ds(j, 1), :] = cand_i[:, None, :]

        @pl.when(j == NKT - 1)
        def _emit():
            s = buf_s_ref[...]
            i = buf_i_ref[...]
            for t in range(TOPK):
                m = jnp.max(s, axis=(1, 2))
                a = jnp.min(jnp.where(s == m[:, None, None], i, IBIG), axis=(1, 2))
                out_s_ref[:, t] = m
                out_i_ref[:, t] = a
                s = jnp.where(i == a[:, None, None], NEG, s)


@jax.jit
def kernel(queries, keys):
    keys_p = jnp.pad(keys, ((0, KPAD - K), (0, 0)))
    qnorm = jnp.linalg.norm(queries, axis=-1, keepdims=True) + 1e-8
    knorm = jnp.linalg.norm(keys_p, axis=-1, keepdims=True) + 1e-8
    grid = (Q // BQ, 2, NKT)
    out_s, out_i = pl.pallas_call(
        _knn_kernel,
        grid=grid,
        in_specs=[
            pl.BlockSpec((BQ, D), lambda i, s, j: (i, 0)),
            pl.BlockSpec((BK, D), lambda i, s, j: (j, 0)),
            pl.BlockSpec((BQ, 1), lambda i, s, j: (i, 0)),
            pl.BlockSpec((BK, 1), lambda i, s, j: (j, 0)),
        ],
        out_specs=[
            pl.BlockSpec((BQ, TOPK), lambda i, s, j: (i, 0)),
            pl.BlockSpec((BQ, TOPK), lambda i, s, j: (i, 0)),
        ],
        out_shape=[
            jax.ShapeDtypeStruct((Q, TOPK), jnp.float32),
            jax.ShapeDtypeStruct((Q, TOPK), jnp.int32),
        ],
        scratch_shapes=[
            pltpu.VMEM((BQ, D), jnp.float32),
            pltpu.VMEM((BQ, BK), jnp.float32),
            pltpu.VMEM((BQ, NKT, 1), jnp.float32),
            pltpu.VMEM((BQ, 1), jnp.float32),
            pltpu.VMEM((BQ, NKT, TOPK), jnp.float32),
            pltpu.VMEM((BQ, NKT, TOPK), jnp.int32),
        ],
    )(queries, keys_p, qnorm, knorm)
    return out_s, out_i


# speedup vs baseline: 1.9172x; 1.1077x over previous
import jax
import jax.numpy as jnp
from jax.experimental import pallas as pl
from jax.experimental.pallas import tpu as pltpu

Q = 1024
K = 100000
D = 128
TOPK = 8
BQ = 256
BK = 4096
KPAD = 102400
NKT = KPAD // BK
IBIG = 2**30
NEG = -jnp.inf


def _tree_max(s):
    w = s.shape[1]
    while w > 128:
        h = w // 2
        s = jnp.maximum(s[:, :h], s[:, h:w])
        w = h
    return jnp.max(s, axis=1)


def _tree_max_arg(s, col):
    w = s.shape[1]
    while w > 128:
        h = w // 2
        lo, hi = s[:, :h], s[:, h:w]
        cl, ch = col[:, :h], col[:, h:w]
        cmp = lo >= hi
        s = jnp.where(cmp, lo, hi)
        col = jnp.where(cmp, cl, ch)
        w = h
    m = jnp.max(s, axis=1)
    a = jnp.min(jnp.where(s == m[:, None], col, IBIG), axis=1)
    return m, a


def _knn_kernel(q_ref, k_ref, qn2_ref, kn2_ref, out_s_ref, out_i_ref,
                qn_ref, sims_ref, tmax_ref, tau_ref, buf_s_ref, buf_i_ref):
    sweep = pl.program_id(1)
    j = pl.program_id(2)

    @pl.when((sweep == 0) & (j == 0))
    def _init():
        qn_ref[...] = q_ref[...] / qn2_ref[...]

    kn = k_ref[...] / kn2_ref[...]
    sims = jax.lax.dot_general(
        qn_ref[...], kn, (((1,), (1,)), ((), ())),
        preferred_element_type=jnp.float32,
        precision=jax.lax.Precision.DEFAULT,
    )
    col = jax.lax.broadcasted_iota(jnp.int32, (BQ, BK), 1) + j * BK
    sims = jnp.where(col < K, sims, NEG)

    @pl.when(sweep == 0)
    def _sweep1():
        tmax_ref[:, pl.ds(j, 1), :] = _tree_max(sims)[:, None, None]

    @pl.when(sweep == 1)
    def _sweep2():
        @pl.when(j == 0)
        def _tau():
            v = tmax_ref[...]
            for _ in range(TOPK - 1):
                mv = jnp.max(v, axis=(1, 2))
                v = jnp.where(v == mv[:, None, None], NEG, v)
            tau_ref[...] = jnp.max(v, axis=(1, 2))[:, None]

        sims_ref[...] = sims
        tau = tau_ref[...][:, 0]
        m0, a0 = _tree_max_arg(sims, col)
        lane8 = jax.lax.broadcasted_iota(jnp.int32, (BQ, TOPK), 1)
        cand_s0 = jnp.full((BQ, TOPK), NEG, dtype=jnp.float32)
        cand_i0 = jnp.full((BQ, TOPK), IBIG, dtype=jnp.int32)

        def cond(carry):
            t, _, _, m, _ = carry
            return (t < TOPK) & jnp.any(m >= tau)

        def body(carry):
            t, cand_s, cand_i, m, a = carry
            upd = (m >= tau)[:, None] & (lane8 == t)
            cand_s = jnp.where(upd, m[:, None], cand_s)
            cand_i = jnp.where(upd, a[:, None], cand_i)
            sims_ref[...] = jnp.where(col == a[:, None], NEG, sims_ref[...])
            m, a = _tree_max_arg(sims_ref[...], col)
            return t + 1, cand_s, cand_i, m, a

        _, cand_s, cand_i, _, _ = jax.lax.while_loop(
            cond, body, (jnp.int32(0), cand_s0, cand_i0, m0, a0))
        buf_s_ref[:, pl.ds(j, 1), :] = cand_s[:, None, :]
        buf_i_ref[:, pl.ds(j, 1), :] = cand_i[:, None, :]

        @pl.when(j == NKT - 1)
        def _emit():
            s = buf_s_ref[...]
            i = buf_i_ref[...]
            for t in range(TOPK):
                m = jnp.max(s, axis=(1, 2))
                a = jnp.min(jnp.where(s == m[:, None, None], i, IBIG), axis=(1, 2))
                out_s_ref[:, t] = m
                out_i_ref[:, t] = a
                s = jnp.where(i == a[:, None, None], NEG, s)


@jax.jit
def kernel(queries, keys):
    keys_p = jnp.pad(keys, ((0, KPAD - K), (0, 0)))
    qnorm = jnp.linalg.norm(queries, axis=-1, keepdims=True) + 1e-8
    knorm = jnp.linalg.norm(keys_p, axis=-1, keepdims=True) + 1e-8
    grid = (Q // BQ, 2, NKT)
    out_s, out_i = pl.pallas_call(
        _knn_kernel,
        grid=grid,
        in_specs=[
            pl.BlockSpec((BQ, D), lambda i, s, j: (i, 0)),
            pl.BlockSpec((BK, D), lambda i, s, j: (j, 0)),
            pl.BlockSpec((BQ, 1), lambda i, s, j: (i, 0)),
            pl.BlockSpec((BK, 1), lambda i, s, j: (j, 0)),
        ],
        out_specs=[
            pl.BlockSpec((BQ, TOPK), lambda i, s, j: (i, 0)),
            pl.BlockSpec((BQ, TOPK), lambda i, s, j: (i, 0)),
        ],
        out_shape=[
            jax.ShapeDtypeStruct((Q, TOPK), jnp.float32),
            jax.ShapeDtypeStruct((Q, TOPK), jnp.int32),
        ],
        scratch_shapes=[
            pltpu.VMEM((BQ, D), jnp.float32),
            pltpu.VMEM((BQ, BK), jnp.float32),
            pltpu.VMEM((BQ, NKT, 1), jnp.float32),
            pltpu.VMEM((BQ, 1), jnp.float32),
            pltpu.VMEM((BQ, NKT, TOPK), jnp.float32),
            pltpu.VMEM((BQ, NKT, TOPK), jnp.int32),
        ],
    )(queries, keys_p, qnorm, knorm)
    return out_s, out_i


# speedup vs baseline: 2.2255x; 1.1608x over previous
import jax
import jax.numpy as jnp
from jax.experimental import pallas as pl
from jax.experimental.pallas import tpu as pltpu

Q = 1024
K = 100000
D = 128
TOPK = 8
BQ = 256
BK = 4096
KPAD = 102400
NKT = KPAD // BK
NCH = 8
CHW = BK // NCH
NCAND = NKT * TOPK
IBIG = 2**30
NEG = -jnp.inf


def _tree_max(s):
    w = s.shape[1]
    while w > 128:
        h = w // 2
        s = jnp.maximum(s[:, :h], s[:, h:w])
        w = h
    return jnp.max(s, axis=1)


def _tree_max_arg(s, col):
    w = s.shape[1]
    while w > 128:
        h = w // 2
        lo, hi = s[:, :h], s[:, h:w]
        cl, ch = col[:, :h], col[:, h:w]
        cmp = lo >= hi
        s = jnp.where(cmp, lo, hi)
        col = jnp.where(cmp, cl, ch)
        w = h
    m = jnp.max(s, axis=1)
    a = jnp.min(jnp.where(s == m[:, None], col, IBIG), axis=1)
    return m, a


def _knn_kernel(q_ref, k_ref, qn2_ref, kn2_ref, out_s_ref, out_i_ref,
                qn_ref, sims_ref, cmax_ref, tau_ref, buf_s_ref, buf_i_ref):
    sweep = pl.program_id(1)
    j = pl.program_id(2)

    @pl.when((sweep == 0) & (j == 0))
    def _init():
        qn_ref[...] = q_ref[...] / qn2_ref[...]

    kn = k_ref[...] / kn2_ref[...]
    sims = jax.lax.dot_general(
        qn_ref[...], kn, (((1,), (1,)), ((), ())),
        preferred_element_type=jnp.float32,
        precision=jax.lax.Precision.DEFAULT,
    )
    col = jax.lax.broadcasted_iota(jnp.int32, (BQ, BK), 1) + j * BK
    sims = jnp.where(col < K, sims, NEG)

    @pl.when(sweep == 0)
    def _sweep1():
        cm = cmax_ref[...]
        clane = jax.lax.broadcasted_iota(jnp.int32, (BQ, NKT * NCH), 1)
        for c in range(NCH):
            mc = _tree_max(sims[:, c * CHW:(c + 1) * CHW])
            cm = jnp.where(clane == j * NCH + c, mc[:, None], cm)
        cmax_ref[...] = cm

    @pl.when(sweep == 1)
    def _sweep2():
        @pl.when(j == 0)
        def _tau():
            v = cmax_ref[...]
            for _ in range(TOPK - 1):
                mv = jnp.max(v, axis=1)
                v = jnp.where(v == mv[:, None], NEG, v)
            tau_ref[...] = jnp.max(v, axis=1)[:, None]
            buf_s_ref[...] = jnp.full((BQ, NCAND), NEG, dtype=jnp.float32)
            buf_i_ref[...] = jnp.full((BQ, NCAND), IBIG, dtype=jnp.int32)

        sims_ref[...] = sims
        tau = tau_ref[...][:, 0]
        m0, a0 = _tree_max_arg(sims, col)
        blane = jax.lax.broadcasted_iota(jnp.int32, (BQ, NCAND), 1)

        def cond(carry):
            t, m, _ = carry
            return (t < TOPK) & jnp.any(m >= tau)

        def body(carry):
            t, m, a = carry
            slot = j * TOPK + t
            put = (m >= tau)[:, None] & (blane == slot)
            buf_s_ref[...] = jnp.where(put, m[:, None], buf_s_ref[...])
            buf_i_ref[...] = jnp.where(put, a[:, None], buf_i_ref[...])
            sims_ref[...] = jnp.where(col == a[:, None], NEG, sims_ref[...])
            m, a = _tree_max_arg(sims_ref[...], col)
            return t + 1, m, a

        jax.lax.while_loop(cond, body, (jnp.int32(0), m0, a0))

        @pl.when(j == NKT - 1)
        def _emit():
            s = buf_s_ref[...]
            i = buf_i_ref[...]
            for t in range(TOPK):
                m = jnp.max(s, axis=1)
                a = jnp.min(jnp.where(s == m[:, None], i, IBIG), axis=1)
                out_s_ref[:, t] = m
                out_i_ref[:, t] = a
                s = jnp.where(i == a[:, None], NEG, s)


@jax.jit
def kernel(queries, keys):
    keys_p = jnp.pad(keys, ((0, KPAD - K), (0, 0)))
    qnorm = jnp.linalg.norm(queries, axis=-1, keepdims=True) + 1e-8
    knorm = jnp.linalg.norm(keys_p, axis=-1, keepdims=True) + 1e-8
    grid = (Q // BQ, 2, NKT)
    out_s, out_i = pl.pallas_call(
        _knn_kernel,
        grid=grid,
        in_specs=[
            pl.BlockSpec((BQ, D), lambda i, s, j: (i, 0)),
            pl.BlockSpec((BK, D), lambda i, s, j: (j, 0)),
            pl.BlockSpec((BQ, 1), lambda i, s, j: (i, 0)),
            pl.BlockSpec((BK, 1), lambda i, s, j: (j, 0)),
        ],
        out_specs=[
            pl.BlockSpec((BQ, TOPK), lambda i, s, j: (i, 0)),
            pl.BlockSpec((BQ, TOPK), lambda i, s, j: (i, 0)),
        ],
        out_shape=[
            jax.ShapeDtypeStruct((Q, TOPK), jnp.float32),
            jax.ShapeDtypeStruct((Q, TOPK), jnp.int32),
        ],
        scratch_shapes=[
            pltpu.VMEM((BQ, D), jnp.float32),
            pltpu.VMEM((BQ, BK), jnp.float32),
            pltpu.VMEM((BQ, NKT * NCH), jnp.float32),
            pltpu.VMEM((BQ, 1), jnp.float32),
            pltpu.VMEM((BQ, NCAND), jnp.float32),
            pltpu.VMEM((BQ, NCAND), jnp.int32),
        ],
    )(queries, keys_p, qnorm, knorm)
    return out_s, out_i
